# Initial kernel scaffold; baseline (speedup 1.0000x reference)
import jax
import jax.numpy as jnp
from jax.experimental import pallas as pl


def kernel(inputs, params):
    raise NotImplementedError("write your pallas kernel here")


# speedup vs baseline: 16.6097x; 16.6097x over previous
import functools

import jax
import jax.numpy as jnp
import numpy as np
from jax import lax
from jax.experimental import pallas as pl
from jax.experimental.pallas import tpu as pltpu
from jax.experimental.pallas import tpu_sc as plsc

_RES = [16, 23, 32, 46, 64, 92, 128, 184, 256, 368, 512, 736]
_LOG2_HASH = 19
_HASH_SIZE = 1 << _LOG2_HASH
_P1 = np.int32(np.uint32(2654435761).astype(np.int64) - (1 << 32))
_P2 = np.int32(805459861)

_SIZES = [int(np.ceil(min(_HASH_SIZE, r ** 3) / 8) * 8) for r in _RES]
_OFFS = np.concatenate([[0], np.cumsum(_SIZES)]).astype(np.int64)
_HASHED = [r ** 3 > _HASH_SIZE for r in _RES]

_NWORDS = [int(np.ceil((s // 8) / 8) * 8) for s in _SIZES]
_WOFF = np.concatenate([[0], np.cumsum(_NWORDS)]).astype(np.int64)
_TOT_WORDS = int(_WOFF[-1])
_PACK_ROWS = 128
_TOT_WORDS_PAD = int(np.ceil(_TOT_WORDS / (8 * _PACK_ROWS)) * 8 * _PACK_ROWS)
_NLEV = len(_RES)
_TAB_MAX = max(_NWORDS)

_BLK = 1024
_LANES = 16


def _pack_body(pr_ref, out_ref):
    w = jnp.zeros((8, _PACK_ROWS), jnp.int32)
    for m in range(16):
        w = w | ((pr_ref[m] >= 0.0).astype(jnp.int32) << m)
    out_ref[...] = w


def _pack_table(params):
    segs = []
    for l in range(_NLEV):
        seg = params[int(_OFFS[l]):int(_OFFS[l + 1])]
        pad = _NWORDS[l] * 8 - seg.shape[0]
        if pad:
            seg = jnp.pad(seg, ((0, pad), (0, 0)))
        segs.append(seg)
    tail = _TOT_WORDS_PAD * 8 - _TOT_WORDS * 8
    if tail:
        segs.append(jnp.zeros((tail, 2), params.dtype))
    params_p = jnp.concatenate(segs, axis=0)
    planes = params_p.reshape(_TOT_WORDS_PAD, 16).T
    nrow = _TOT_WORDS_PAD // _PACK_ROWS
    planes = planes.reshape(16, nrow, _PACK_ROWS)
    grid = nrow // 8
    packed = pl.pallas_call(
        _pack_body,
        grid=(grid,),
        in_specs=[pl.BlockSpec((16, 8, _PACK_ROWS), lambda i: (0, i, 0))],
        out_specs=pl.BlockSpec((8, _PACK_ROWS), lambda i: (i, 0)),
        out_shape=jax.ShapeDtypeStruct((nrow, _PACK_ROWS), jnp.int32),
    )(planes)
    return packed.reshape(_TOT_WORDS_PAD)


def _sc_grid_encode(npad, nblk):
    mesh = plsc.VectorSubcoreMesh(core_axis_name="c", subcore_axis_name="s")
    chunk = nblk * _BLK

    @functools.partial(
        pl.kernel,
        mesh=mesh,
        compiler_params=pltpu.CompilerParams(needs_layout_passes=False),
        out_type=jax.ShapeDtypeStruct((2 * _NLEV, npad), jnp.float32),
        scratch_types=[
            pltpu.VMEM((_TAB_MAX,), jnp.int32),
            pltpu.VMEM((_BLK,), jnp.float32),
            pltpu.VMEM((_BLK,), jnp.float32),
            pltpu.VMEM((_BLK,), jnp.float32),
            pltpu.VMEM((2, _BLK), jnp.float32),
        ],
    )
    def body(x_hbm, y_hbm, z_hbm, tab_hbm, out_hbm, tab_v, x_v, y_v, z_v, o_v):
        wid = lax.axis_index("s") * 2 + lax.axis_index("c")
        base0 = wid * chunk
        for l in range(_NLEV):
            res = _RES[l]
            nw = _NWORDS[l]
            pltpu.sync_copy(tab_hbm.at[pl.ds(int(_WOFF[l]), nw)],
                            tab_v.at[pl.ds(0, nw)])

            def blk_body(b, carry, l=l, res=res):
                gbase = base0 + b * _BLK
                pltpu.sync_copy(x_hbm.at[pl.ds(gbase, _BLK)], x_v)
                pltpu.sync_copy(y_hbm.at[pl.ds(gbase, _BLK)], y_v)
                pltpu.sync_copy(z_hbm.at[pl.ds(gbase, _BLK)], z_v)

                def vec_body(i, carry2, res=res):
                    s = pl.ds(i * _LANES, _LANES)
                    rm1 = jnp.float32(res - 1.0)
                    px = x_v[s] * rm1
                    py = y_v[s] * rm1
                    pz = z_v[s] * rm1
                    bx = px.astype(jnp.int32)
                    by = py.astype(jnp.int32)
                    bz = pz.astype(jnp.int32)
                    fx = px - bx.astype(jnp.float32)
                    fy = py - by.astype(jnp.float32)
                    fz = pz - bz.astype(jnp.float32)
                    wx = (1.0 - fx, fx)
                    wy = (1.0 - fy, fy)
                    wz = (1.0 - fz, fz)
                    if _HASHED[l]:
                        xs = (bx, bx + 1)
                        ys = (by * _P1, by * _P1 + _P1)
                        zs = (bz * _P2, bz * _P2 + _P2)
                    else:
                        r = np.int32(res)
                        r2 = np.int32(res * res)
                        xs = (bx, bx + 1)
                        ys = (by * r, by * r + r)
                        zs = (bz * r2, bz * r2 + r2)
                    o0 = jnp.zeros((_LANES,), jnp.float32)
                    o1 = jnp.zeros((_LANES,), jnp.float32)
                    for c in range(8):
                        i0 = c & 1
                        i1 = (c >> 1) & 1
                        i2 = (c >> 2) & 1
                        if _HASHED[l]:
                            idx = (xs[i0] ^ ys[i1] ^ zs[i2]) & np.int32(_HASH_SIZE - 1)
                        else:
                            idx = xs[i0] + ys[i1] + zs[i2]
                        word = plsc.load_gather(tab_v, [idx >> 3])
                        t = word >> ((idx & 7) << 1)
                        w = wx[i0] * wy[i1] * wz[i2]
                        o0 = o0 + jnp.where((t & 1) != 0, w, -w)
                        o1 = o1 + jnp.where((t & 2) != 0, w, -w)
                    o_v[0, s] = o0
                    o_v[1, s] = o1
                    return carry2

                lax.fori_loop(0, _BLK // _LANES, vec_body, 0)
                pltpu.sync_copy(
                    o_v, out_hbm.at[pl.ds(2 * l, 2), pl.ds(gbase, _BLK)])
                return carry

            lax.fori_loop(0, nblk, blk_body, 0)

    return body


@jax.jit
def kernel(inputs, params):
    n = inputs.shape[0]
    info = plsc.get_sparse_core_info()
    ntiles = info.num_cores * info.num_subcores
    nblk = int(np.ceil(n / (ntiles * _BLK)))
    npad = ntiles * nblk * _BLK
    pts = jnp.pad(inputs, ((0, npad - n), (0, 0))).T
    tab = _pack_table(params)
    out_t = _sc_grid_encode(npad, nblk)(pts[0], pts[1], pts[2], tab)
    return out_t[:, :n].T


# speedup vs baseline: 17.6291x; 1.0614x over previous
import functools

import jax
import jax.numpy as jnp
import numpy as np
from jax import lax
from jax.experimental import pallas as pl
from jax.experimental.pallas import tpu as pltpu
from jax.experimental.pallas import tpu_sc as plsc

_RES = [16, 23, 32, 46, 64, 92, 128, 184, 256, 368, 512, 736]
_LOG2_HASH = 19
_HASH_SIZE = 1 << _LOG2_HASH
_P1 = np.int32(np.uint32(2654435761).astype(np.int64) - (1 << 32))
_P2 = np.int32(805459861)

_SIZES = [int(np.ceil(min(_HASH_SIZE, r ** 3) / 8) * 8) for r in _RES]
_OFFS = np.concatenate([[0], np.cumsum(_SIZES)]).astype(np.int64)
_HASHED = [r ** 3 > _HASH_SIZE for r in _RES]

_NWORDS = [int(np.ceil((s // 8) / 8) * 8) for s in _SIZES]
_WOFF = np.concatenate([[0], np.cumsum(_NWORDS)]).astype(np.int64)
_TOT_WORDS = int(_WOFF[-1])
_PACK_ROWS = 128
_TOT_WORDS_PAD = int(np.ceil(_TOT_WORDS / (8 * _PACK_ROWS)) * 8 * _PACK_ROWS)
_NLEV = len(_RES)
_TAB_MAX = max(_NWORDS)

_BLK = 8192
_LANES = 16
_UNROLL = 2


def _pack_body(pr_ref, out_ref):
    w = jnp.zeros((8, _PACK_ROWS), jnp.int32)
    for m in range(16):
        w = w | ((pr_ref[m] >= 0.0).astype(jnp.int32) << m)
    out_ref[...] = w


def _pack_table(params):
    segs = []
    for l in range(_NLEV):
        seg = params[int(_OFFS[l]):int(_OFFS[l + 1])]
        pad = _NWORDS[l] * 8 - seg.shape[0]
        if pad:
            seg = jnp.pad(seg, ((0, pad), (0, 0)))
        segs.append(seg)
    tail = _TOT_WORDS_PAD * 8 - _TOT_WORDS * 8
    if tail:
        segs.append(jnp.zeros((tail, 2), params.dtype))
    params_p = jnp.concatenate(segs, axis=0)
    planes = params_p.reshape(_TOT_WORDS_PAD, 16).T
    nrow = _TOT_WORDS_PAD // _PACK_ROWS
    planes = planes.reshape(16, nrow, _PACK_ROWS)
    grid = nrow // 8
    packed = pl.pallas_call(
        _pack_body,
        grid=(grid,),
        in_specs=[pl.BlockSpec((16, 8, _PACK_ROWS), lambda i: (0, i, 0))],
        out_specs=pl.BlockSpec((8, _PACK_ROWS), lambda i: (i, 0)),
        out_shape=jax.ShapeDtypeStruct((nrow, _PACK_ROWS), jnp.int32),
    )(planes)
    return packed.reshape(_TOT_WORDS_PAD)


def _sc_grid_encode(npad, nblk):
    mesh = plsc.VectorSubcoreMesh(core_axis_name="c", subcore_axis_name="s")
    chunk = nblk * _BLK

    @functools.partial(
        pl.kernel,
        mesh=mesh,
        compiler_params=pltpu.CompilerParams(needs_layout_passes=False),
        out_type=jax.ShapeDtypeStruct((2 * _NLEV, npad), jnp.float32),
        scratch_types=[
            pltpu.VMEM((_TAB_MAX,), jnp.int32),
            pltpu.VMEM((_BLK,), jnp.float32),
            pltpu.VMEM((_BLK,), jnp.float32),
            pltpu.VMEM((_BLK,), jnp.float32),
            pltpu.VMEM((2, _BLK), jnp.float32),
        ],
    )
    def body(x_hbm, y_hbm, z_hbm, tab_hbm, out_hbm, tab_v, x_v, y_v, z_v, o_v):
        wid = lax.axis_index("s") * 2 + lax.axis_index("c")
        base0 = wid * chunk
        for l in range(_NLEV):
            res = _RES[l]
            nw = _NWORDS[l]
            pltpu.sync_copy(tab_hbm.at[pl.ds(int(_WOFF[l]), nw)],
                            tab_v.at[pl.ds(0, nw)])

            def blk_body(b, carry, l=l, res=res):
                gbase = base0 + b * _BLK
                pltpu.sync_copy(x_hbm.at[pl.ds(gbase, _BLK)], x_v)
                pltpu.sync_copy(y_hbm.at[pl.ds(gbase, _BLK)], y_v)
                pltpu.sync_copy(z_hbm.at[pl.ds(gbase, _BLK)], z_v)

                def vec_body(i, carry2, res=res):
                  for u in range(_UNROLL):
                    s = pl.ds((i * _UNROLL + u) * _LANES, _LANES)
                    rm1 = jnp.float32(res - 1.0)
                    px = x_v[s] * rm1
                    py = y_v[s] * rm1
                    pz = z_v[s] * rm1
                    bx = px.astype(jnp.int32)
                    by = py.astype(jnp.int32)
                    bz = pz.astype(jnp.int32)
                    fx = px - bx.astype(jnp.float32)
                    fy = py - by.astype(jnp.float32)
                    fz = pz - bz.astype(jnp.float32)
                    wx = (1.0 - fx, fx)
                    wy = (1.0 - fy, fy)
                    wz = (1.0 - fz, fz)
                    if _HASHED[l]:
                        xs = (bx, bx + 1)
                        ys = (by * _P1, by * _P1 + _P1)
                        zs = (bz * _P2, bz * _P2 + _P2)
                    else:
                        r = np.int32(res)
                        r2 = np.int32(res * res)
                        xs = (bx, bx + 1)
                        ys = (by * r, by * r + r)
                        zs = (bz * r2, bz * r2 + r2)
                    o0 = jnp.zeros((_LANES,), jnp.float32)
                    o1 = jnp.zeros((_LANES,), jnp.float32)
                    for c in range(8):
                        i0 = c & 1
                        i1 = (c >> 1) & 1
                        i2 = (c >> 2) & 1
                        if _HASHED[l]:
                            idx = (xs[i0] ^ ys[i1] ^ zs[i2]) & np.int32(_HASH_SIZE - 1)
                        else:
                            idx = xs[i0] + ys[i1] + zs[i2]
                        word = plsc.load_gather(tab_v, [idx >> 3])
                        t = word >> ((idx & 7) << 1)
                        w = wx[i0] * wy[i1] * wz[i2]
                        o0 = o0 + jnp.where((t & 1) != 0, w, -w)
                        o1 = o1 + jnp.where((t & 2) != 0, w, -w)
                    o_v[0, s] = o0
                    o_v[1, s] = o1
                  return carry2

                lax.fori_loop(0, _BLK // (_UNROLL * _LANES), vec_body, 0)
                pltpu.sync_copy(
                    o_v, out_hbm.at[pl.ds(2 * l, 2), pl.ds(gbase, _BLK)])
                return carry

            lax.fori_loop(0, nblk, blk_body, 0)

    return body


@jax.jit
def kernel(inputs, params):
    n = inputs.shape[0]
    info = plsc.get_sparse_core_info()
    ntiles = info.num_cores * info.num_subcores
    nblk = int(np.ceil(n / (ntiles * _BLK)))
    npad = ntiles * nblk * _BLK
    pts = jnp.pad(inputs, ((0, npad - n), (0, 0))).T
    tab = _pack_table(params)
    out_t = _sc_grid_encode(npad, nblk)(pts[0], pts[1], pts[2], tab)
    return out_t[:, :n].T
